# baseline (speedup 1.0000x reference)
import functools
import numpy as np
import jax
import jax.numpy as jnp
from jax import lax
from jax.experimental import pallas as pl
from jax.experimental.pallas import tpu as pltpu
from jax.experimental.pallas import tpu_sc as plsc

FEAT_STRIDE = 16
NUM_ANCHORS = 9
PRE_NMS_TOPN = 6000
POST_NMS_TOPN = 300
NMS_THRESH = 0.7
MIN_SIZE = 16.0
H, W = 50, 80
N = H * W * NUM_ANCHORS
ROWS = 288
NPAD = ROWS * 128
NEG = -1e9
DN = 6144
ROWS2 = DN // 128

NW = 32
GPW = DN // NW
VECS = NPAD // 16
NBLK = 16
VPB = VECS // NBLK


def _whctrs_k(anchor):
    w = anchor[2] - anchor[0] + 1.0
    h = anchor[3] - anchor[1] + 1.0
    return w, h, anchor[0] + 0.5 * (w - 1), anchor[1] + 0.5 * (h - 1)


def _mkanchors_k(ws, hs, x_ctr, y_ctr):
    ws = ws[:, np.newaxis]
    hs = hs[:, np.newaxis]
    return np.hstack((x_ctr - 0.5 * (ws - 1), y_ctr - 0.5 * (hs - 1),
                      x_ctr + 0.5 * (ws - 1), y_ctr + 0.5 * (hs - 1)))


def _gen_anchor_table(base_size=16, ratios=np.array([0.5, 1.0, 2.0]),
                      scales=np.array([8.0, 16.0, 32.0])):
    base_anchor = np.array([1.0, 1.0, base_size, base_size]) - 1.0
    w, h, x_ctr, y_ctr = _whctrs_k(base_anchor)
    ws0 = np.round(np.sqrt((w * h) / ratios))
    hs0 = np.round(ws0 * ratios)
    ratio_anchors = _mkanchors_k(ws0, hs0, x_ctr, y_ctr)
    outs = []
    for i in range(ratio_anchors.shape[0]):
        w, h, x_ctr, y_ctr = _whctrs_k(ratio_anchors[i, :])
        outs.append(_mkanchors_k(w * scales, h * scales, x_ctr, y_ctr))
    return np.vstack(outs).astype(np.float32)


def _anchor_consts():
    base = _gen_anchor_table().astype(np.float64)
    sx = (np.arange(W) * FEAT_STRIDE).astype(np.float64)
    sy = (np.arange(H) * FEAT_STRIDE).astype(np.float64)
    SX, SY = np.meshgrid(sx, sy)
    shifts = np.stack([SX.ravel(), SY.ravel(), SX.ravel(), SY.ravel()], axis=1)
    anchors = (base[None, :, :] + shifts[:, None, :]).reshape(-1, 4)
    widths = anchors[:, 2] - anchors[:, 0] + 1.0
    heights = anchors[:, 3] - anchors[:, 1] + 1.0
    ctr_x = anchors[:, 0] + 0.5 * widths
    ctr_y = anchors[:, 1] + 0.5 * heights

    def padr(v):
        out = np.zeros((NPAD,), np.float32)
        out[:N] = v.astype(np.float32)
        return out.reshape(ROWS, 128)

    return padr(widths), padr(heights), padr(ctr_x), padr(ctr_y)


_WA, _HA, _CXA, _CYA = _anchor_consts()


def _decode_body(sc_ref, dx_ref, dy_ref, dw_ref, dh_ref, wa_ref, ha_ref,
                 cxa_ref, cya_ref, im_ref, sg_ref, u8_ref, ones8_ref,
                 m288_ref, b8_ref, wg_ref, x1o, y1o, x2o, y2o, aro, so,
                 rank_o):
    im_h = im_ref[0, 0]
    im_w = im_ref[0, 1]
    im_scale = im_ref[0, 2]

    wa = wa_ref[...]
    ha = ha_ref[...]
    dw = jnp.clip(dw_ref[...], -10.0, 10.0)
    dh = jnp.clip(dh_ref[...], -10.0, 10.0)
    pcx = dx_ref[...] * wa + cxa_ref[...]
    pcy = dy_ref[...] * ha + cya_ref[...]
    pw = jnp.exp(dw) * wa
    ph = jnp.exp(dh) * ha
    x1 = jnp.clip(pcx - 0.5 * pw, 0.0, im_w - 1.0)
    y1 = jnp.clip(pcy - 0.5 * ph, 0.0, im_h - 1.0)
    x2 = jnp.clip(pcx + 0.5 * pw, 0.0, im_w - 1.0)
    y2 = jnp.clip(pcy + 0.5 * ph, 0.0, im_h - 1.0)
    ws = x2 - x1 + 1.0
    hs = y2 - y1 + 1.0
    min_sz = MIN_SIZE * im_scale
    keep = (ws >= min_sz) & (hs >= min_sz)

    ri = lax.broadcasted_iota(jnp.int32, (ROWS, 128), 0)
    ci = lax.broadcasted_iota(jnp.int32, (ROWS, 128), 1)
    n_i = ri * 128 + ci
    valid = n_i < N

    s0 = jnp.where(keep, sc_ref[...], jnp.float32(NEG))
    s0 = jnp.where(valid, s0, -jnp.inf)

    ks = lax.bitcast_convert_type(s0, jnp.int32)
    key_i = ks ^ ((ks >> 31) & jnp.int32(0x7FFFFFFF))
    ku = lax.bitcast_convert_type(key_i, jnp.uint32) ^ jnp.uint32(0x80000000)

    K = jnp.float32(PRE_NMS_TOPN)

    def tau_step(t, prefix):
        b = (31 - t).astype(jnp.uint32)
        cand = prefix | (jnp.uint32(1) << b)
        cnt = jnp.sum(jnp.where(ku >= cand, 1.0, 0.0))
        return jnp.where(cnt >= K, cand, prefix)

    tau = lax.fori_loop(0, 32, tau_step, jnp.uint32(0))

    c_gt = jnp.sum(jnp.where(ku > tau, 1.0, 0.0))
    needed = K - c_gt
    eqm = ku == tau

    def cut_step(t, prefix):
        cand = prefix | (jnp.int32(1) << (16 - t))
        cnt = jnp.sum(jnp.where(eqm & (n_i < cand), 1.0, 0.0))
        return jnp.where(cnt < needed, cand, prefix)

    tcut = lax.fori_loop(0, 17, cut_step, jnp.int32(0))
    elig = (ku > tau) | (eqm & (n_i <= tcut) & (needed >= 1.0))

    x1o[...] = x1
    y1o[...] = y1
    x2o[...] = x2
    y2o[...] = y2
    aro[...] = ws * hs
    so[...] = jnp.where(elig, s0, -jnp.inf)

    eligf = jnp.where(elig, 1.0, 0.0)
    hi = jax.lax.Precision.HIGHEST
    g8 = jax.lax.dot_general(eligf, sg_ref[...], (((1,), (0,)), ((), ())),
                             precision=hi)
    p_in = jax.lax.dot_general(g8, u8_ref[...], (((1,), (0,)), ((), ())),
                               precision=hi)
    rsum = jax.lax.dot_general(g8, ones8_ref[...], (((1,), (0,)), ((), ())),
                               precision=hi)
    roff = jax.lax.dot_general(m288_ref[...], rsum, (((1,), (0,)), ((), ())),
                               precision=hi)
    cum8 = roff + p_in
    cum_el = jax.lax.dot_general(cum8, b8_ref[...], (((1,), (0,)), ((), ())),
                                 precision=hi)
    p2 = jax.lax.dot_general(eligf, wg_ref[...], (((1,), (0,)), ((), ())),
                             precision=hi)
    rank = (cum_el + p2).astype(jnp.int32)
    rank_o[...] = jnp.where(elig, rank, jnp.int32(DN) + n_i)


def _sel_consts():
    sg = np.zeros((128, 8), np.float32)
    for l in range(128):
        sg[l, l // 16] = 1.0
    u8 = np.triu(np.ones((8, 8), np.float32), 1)
    ones8 = np.ones((8, 1), np.float32)
    m288 = np.tril(np.ones((288, 288), np.float32), -1)
    b8 = np.zeros((8, 128), np.float32)
    for l in range(128):
        b8[l // 16, l] = 1.0
    wg = np.zeros((128, 128), np.float32)
    for a in range(128):
        for b in range(128):
            if a // 16 == b // 16 and a < b:
                wg[a, b] = 1.0
    return sg, u8, ones8, m288, b8, wg


_SG, _U8, _ONES8, _M288, _B8, _WG = _sel_consts()


def _decode(sc, dx, dy, dw, dh, im_info):
    shp = jax.ShapeDtypeStruct((ROWS, 128), jnp.float32)
    shpi = jax.ShapeDtypeStruct((ROWS, 128), jnp.int32)
    return pl.pallas_call(
        _decode_body,
        out_shape=[shp] * 6 + [shpi],
        in_specs=[pl.BlockSpec(memory_space=pltpu.MemorySpace.VMEM)] * 9
        + [pl.BlockSpec(memory_space=pltpu.MemorySpace.SMEM)]
        + [pl.BlockSpec(memory_space=pltpu.MemorySpace.VMEM)] * 6,
        out_specs=[pl.BlockSpec(memory_space=pltpu.MemorySpace.VMEM)] * 7,
    )(sc, dx, dy, dw, dh, jnp.asarray(_WA), jnp.asarray(_HA),
      jnp.asarray(_CXA), jnp.asarray(_CYA), im_info, jnp.asarray(_SG),
      jnp.asarray(_U8), jnp.asarray(_ONES8), jnp.asarray(_M288),
      jnp.asarray(_B8), jnp.asarray(_WG))


NCHUNK = ROWS // 8
DDN = DN + NPAD


def _compact_body(rank_hbm, x1_hbm, y1_hbm, x2_hbm, y2_hbm, ar_hbm, s_hbm,
                  x1o, y1o, x2o, y2o, aro, so,
                  rnk, b0, b1, b2, b3, b4, b5, sem):
    cid = lax.axis_index("c")
    sid = lax.axis_index("s")
    wid = sid * 2 + cid

    for t in range(2):
        ch = wid + NW * t

        @pl.when(ch < NCHUNK)
        def _(ch=ch):
            r0 = pl.multiple_of(ch * 8, 8)
            pltpu.sync_copy(rank_hbm.at[pl.ds(r0, 8)], rnk)
            pending = []
            for src, dst, buf in ((x1_hbm, x1o, b0), (y1_hbm, y1o, b1),
                                  (x2_hbm, x2o, b2), (y2_hbm, y2o, b3),
                                  (ar_hbm, aro, b4), (s_hbm, so, b5)):
                pltpu.sync_copy(src.at[pl.ds(r0, 8)], buf)
                for j in range(8):
                    pending.append(
                        pltpu.async_copy(buf.at[j], dst.at[rnk.at[j]], sem))
            for c in pending:
                c.wait()


@functools.lru_cache(maxsize=1)
def _compact_sc():
    return pl.kernel(
        _compact_body,
        mesh=plsc.VectorSubcoreMesh(core_axis_name="c", subcore_axis_name="s"),
        out_type=[jax.ShapeDtypeStruct((DDN,), jnp.float32)] * 6,
        scratch_types=[
            pltpu.VMEM((8, 128), jnp.int32)]
        + [pltpu.VMEM((8, 128), jnp.float32)] * 6
        + [pltpu.SemaphoreType.DMA],
    )


def _nms_body(x1_ref, y1_ref, x2_ref, y2_ref, ar_ref, sc_ref, out_ref, s_ref):
    ri = lax.broadcasted_iota(jnp.int32, (ROWS2, 128), 0)
    ci = lax.broadcasted_iota(jnp.int32, (ROWS2, 128), 1)
    n_i = ri * 128 + ci
    s_ref[...] = jnp.where(n_i < PRE_NMS_TOPN, sc_ref[...], -jnp.inf)

    nf = ri.astype(jnp.float32) * 128.0 + ci.astype(jnp.float32)
    li = lax.broadcasted_iota(jnp.int32, (1, 128), 1)

    def nms_step(step, i0):
        s = s_ref[...]
        m = jnp.max(s)
        idx = jnp.min(jnp.where(s == m, nf, jnp.float32(DN))).astype(jnp.int32)
        i0n = jnp.where(step == 0, idx, i0)
        sel = jnp.where(m == jnp.float32(NEG), i0n, idx)
        r = sel // 128
        c = sel % 128
        lm = li == c
        bx1 = jnp.sum(jnp.where(lm, x1_ref[pl.ds(r, 1), :], 0.0))
        by1 = jnp.sum(jnp.where(lm, y1_ref[pl.ds(r, 1), :], 0.0))
        bx2 = jnp.sum(jnp.where(lm, x2_ref[pl.ds(r, 1), :], 0.0))
        by2 = jnp.sum(jnp.where(lm, y2_ref[pl.ds(r, 1), :], 0.0))
        bar = jnp.sum(jnp.where(lm, ar_ref[pl.ds(r, 1), :], 0.0))
        w = jnp.maximum(0.0, jnp.minimum(bx2, x2_ref[...])
                        - jnp.maximum(bx1, x1_ref[...]) + 1.0)
        h = jnp.maximum(0.0, jnp.minimum(by2, y2_ref[...])
                        - jnp.maximum(by1, y1_ref[...]) + 1.0)
        inter = w * h
        iou = inter / (bar + ar_ref[...] - inter)
        s_ref[...] = jnp.where(iou > jnp.float32(NMS_THRESH),
                               jnp.minimum(s, jnp.float32(NEG)), s)
        rv = jnp.zeros((1, 128), jnp.float32)
        rv = jnp.where(li == 1, bx1, rv)
        rv = jnp.where(li == 2, by1, rv)
        rv = jnp.where(li == 3, bx2, rv)
        rv = jnp.where(li == 4, by2, rv)
        out_ref[pl.ds(step, 1), :] = rv
        return i0n

    lax.fori_loop(0, POST_NMS_TOPN, nms_step, jnp.int32(0))


def _nms(x1c, y1c, x2c, y2c, arc, sc):
    return pl.pallas_call(
        _nms_body,
        out_shape=jax.ShapeDtypeStruct((POST_NMS_TOPN, 128), jnp.float32),
        in_specs=[pl.BlockSpec(memory_space=pltpu.MemorySpace.VMEM)] * 6,
        out_specs=pl.BlockSpec(memory_space=pltpu.MemorySpace.VMEM),
        scratch_shapes=[pltpu.VMEM((ROWS2, 128), jnp.float32)],
    )(x1c, y1c, x2c, y2c, arc, sc)


def kernel(scores, bbox_deltas, im_info):
    sfg = jnp.transpose(scores[0, NUM_ANCHORS:], (1, 2, 0)).reshape(-1)
    dl = jnp.transpose(bbox_deltas[0], (1, 2, 0)).reshape(-1, 4)

    def pad2(v):
        return jnp.concatenate(
            [v, jnp.zeros((NPAD - N,), jnp.float32)]).reshape(ROWS, 128)

    x1, y1, x2, y2, ar, s, rank = _decode(
        pad2(sfg), pad2(dl[:, 0]), pad2(dl[:, 1]), pad2(dl[:, 2]),
        pad2(dl[:, 3]), im_info)
    x1c, y1c, x2c, y2c, arc, sc = _compact_sc()(rank, x1, y1, x2, y2, ar, s)

    def dn(a):
        return a[:DN].reshape(ROWS2, 128)

    out = _nms(dn(x1c), dn(y1c), dn(x2c), dn(y2c), dn(arc), dn(sc))
    return out[:, :5]


# speedup vs baseline: 3.3774x; 3.3774x over previous
import functools
import numpy as np
import jax
import jax.numpy as jnp
from jax import lax
from jax.experimental import pallas as pl
from jax.experimental.pallas import tpu as pltpu
from jax.experimental.pallas import tpu_sc as plsc

FEAT_STRIDE = 16
NUM_ANCHORS = 9
PRE_NMS_TOPN = 6000
POST_NMS_TOPN = 300
NMS_THRESH = 0.7
MIN_SIZE = 16.0
H, W = 50, 80
N = H * W * NUM_ANCHORS
ROWS = 288
NPAD = ROWS * 128
NEG = -1e9
DN = 6144
ROWS2 = DN // 128

NW = 32
GPW = DN // NW
VECS = NPAD // 16
NBLK = 16
VPB = VECS // NBLK


def _whctrs_k(anchor):
    w = anchor[2] - anchor[0] + 1.0
    h = anchor[3] - anchor[1] + 1.0
    return w, h, anchor[0] + 0.5 * (w - 1), anchor[1] + 0.5 * (h - 1)


def _mkanchors_k(ws, hs, x_ctr, y_ctr):
    ws = ws[:, np.newaxis]
    hs = hs[:, np.newaxis]
    return np.hstack((x_ctr - 0.5 * (ws - 1), y_ctr - 0.5 * (hs - 1),
                      x_ctr + 0.5 * (ws - 1), y_ctr + 0.5 * (hs - 1)))


def _gen_anchor_table(base_size=16, ratios=np.array([0.5, 1.0, 2.0]),
                      scales=np.array([8.0, 16.0, 32.0])):
    base_anchor = np.array([1.0, 1.0, base_size, base_size]) - 1.0
    w, h, x_ctr, y_ctr = _whctrs_k(base_anchor)
    ws0 = np.round(np.sqrt((w * h) / ratios))
    hs0 = np.round(ws0 * ratios)
    ratio_anchors = _mkanchors_k(ws0, hs0, x_ctr, y_ctr)
    outs = []
    for i in range(ratio_anchors.shape[0]):
        w, h, x_ctr, y_ctr = _whctrs_k(ratio_anchors[i, :])
        outs.append(_mkanchors_k(w * scales, h * scales, x_ctr, y_ctr))
    return np.vstack(outs).astype(np.float32)


def _anchor_consts():
    base = _gen_anchor_table().astype(np.float64)
    sx = (np.arange(W) * FEAT_STRIDE).astype(np.float64)
    sy = (np.arange(H) * FEAT_STRIDE).astype(np.float64)
    SX, SY = np.meshgrid(sx, sy)
    shifts = np.stack([SX.ravel(), SY.ravel(), SX.ravel(), SY.ravel()], axis=1)
    anchors = (base[None, :, :] + shifts[:, None, :]).reshape(-1, 4)
    widths = anchors[:, 2] - anchors[:, 0] + 1.0
    heights = anchors[:, 3] - anchors[:, 1] + 1.0
    ctr_x = anchors[:, 0] + 0.5 * widths
    ctr_y = anchors[:, 1] + 0.5 * heights

    def padr(v):
        out = np.zeros((NPAD,), np.float32)
        out[:N] = v.astype(np.float32)
        return out.reshape(ROWS, 128)

    return padr(widths), padr(heights), padr(ctr_x), padr(ctr_y)


_WA, _HA, _CXA, _CYA = _anchor_consts()


def _decode_body(sc_ref, dx_ref, dy_ref, dw_ref, dh_ref, wa_ref, ha_ref,
                 cxa_ref, cya_ref, im_ref, sg_ref, u8_ref, ones8_ref,
                 m288_ref, b8_ref, wg_ref, x1o, y1o, x2o, y2o, aro, so,
                 rank_o):
    im_h = im_ref[0, 0]
    im_w = im_ref[0, 1]
    im_scale = im_ref[0, 2]

    wa = wa_ref[...]
    ha = ha_ref[...]
    dw = jnp.clip(dw_ref[...], -10.0, 10.0)
    dh = jnp.clip(dh_ref[...], -10.0, 10.0)
    pcx = dx_ref[...] * wa + cxa_ref[...]
    pcy = dy_ref[...] * ha + cya_ref[...]
    pw = jnp.exp(dw) * wa
    ph = jnp.exp(dh) * ha
    x1 = jnp.clip(pcx - 0.5 * pw, 0.0, im_w - 1.0)
    y1 = jnp.clip(pcy - 0.5 * ph, 0.0, im_h - 1.0)
    x2 = jnp.clip(pcx + 0.5 * pw, 0.0, im_w - 1.0)
    y2 = jnp.clip(pcy + 0.5 * ph, 0.0, im_h - 1.0)
    ws = x2 - x1 + 1.0
    hs = y2 - y1 + 1.0
    min_sz = MIN_SIZE * im_scale
    keep = (ws >= min_sz) & (hs >= min_sz)

    ri = lax.broadcasted_iota(jnp.int32, (ROWS, 128), 0)
    ci = lax.broadcasted_iota(jnp.int32, (ROWS, 128), 1)
    n_i = ri * 128 + ci
    valid = n_i < N

    s0 = jnp.where(keep, sc_ref[...], jnp.float32(NEG))
    s0 = jnp.where(valid, s0, -jnp.inf)

    ks = lax.bitcast_convert_type(s0, jnp.int32)
    key_i = ks ^ ((ks >> 31) & jnp.int32(0x7FFFFFFF))
    ku = lax.bitcast_convert_type(key_i, jnp.uint32) ^ jnp.uint32(0x80000000)

    K = jnp.float32(PRE_NMS_TOPN)

    def tau_step(t, prefix):
        b = (31 - t).astype(jnp.uint32)
        cand = prefix | (jnp.uint32(1) << b)
        cnt = jnp.sum(jnp.where(ku >= cand, 1.0, 0.0))
        return jnp.where(cnt >= K, cand, prefix)

    tau = lax.fori_loop(0, 32, tau_step, jnp.uint32(0))

    c_gt = jnp.sum(jnp.where(ku > tau, 1.0, 0.0))
    needed = K - c_gt
    eqm = ku == tau

    def cut_step(t, prefix):
        cand = prefix | (jnp.int32(1) << (16 - t))
        cnt = jnp.sum(jnp.where(eqm & (n_i < cand), 1.0, 0.0))
        return jnp.where(cnt < needed, cand, prefix)

    tcut = lax.fori_loop(0, 17, cut_step, jnp.int32(0))
    elig = (ku > tau) | (eqm & (n_i <= tcut) & (needed >= 1.0))

    x1o[...] = x1
    y1o[...] = y1
    x2o[...] = x2
    y2o[...] = y2
    aro[...] = ws * hs
    so[...] = jnp.where(elig, s0, -jnp.inf)

    eligf = jnp.where(elig, 1.0, 0.0)
    hi = jax.lax.Precision.HIGHEST
    g8 = jax.lax.dot_general(eligf, sg_ref[...], (((1,), (0,)), ((), ())),
                             precision=hi)
    p_in = jax.lax.dot_general(g8, u8_ref[...], (((1,), (0,)), ((), ())),
                               precision=hi)
    rsum = jax.lax.dot_general(g8, ones8_ref[...], (((1,), (0,)), ((), ())),
                               precision=hi)
    roff = jax.lax.dot_general(m288_ref[...], rsum, (((1,), (0,)), ((), ())),
                               precision=hi)
    cum8 = roff + p_in
    cum_el = jax.lax.dot_general(cum8, b8_ref[...], (((1,), (0,)), ((), ())),
                                 precision=hi)
    p2 = jax.lax.dot_general(eligf, wg_ref[...], (((1,), (0,)), ((), ())),
                             precision=hi)
    rank = (cum_el + p2).astype(jnp.int32)
    rank_o[...] = jnp.where(elig, rank, jnp.int32(DN) + n_i)


def _sel_consts():
    sg = np.zeros((128, 8), np.float32)
    for l in range(128):
        sg[l, l // 16] = 1.0
    u8 = np.triu(np.ones((8, 8), np.float32), 1)
    ones8 = np.ones((8, 1), np.float32)
    m288 = np.tril(np.ones((288, 288), np.float32), -1)
    b8 = np.zeros((8, 128), np.float32)
    for l in range(128):
        b8[l // 16, l] = 1.0
    wg = np.zeros((128, 128), np.float32)
    for a in range(128):
        for b in range(128):
            if a // 16 == b // 16 and a < b:
                wg[a, b] = 1.0
    return sg, u8, ones8, m288, b8, wg


_SG, _U8, _ONES8, _M288, _B8, _WG = _sel_consts()


def _decode(sc, dx, dy, dw, dh, im_info):
    shp = jax.ShapeDtypeStruct((ROWS, 128), jnp.float32)
    shpi = jax.ShapeDtypeStruct((ROWS, 128), jnp.int32)
    return pl.pallas_call(
        _decode_body,
        out_shape=[shp] * 6 + [shpi],
        in_specs=[pl.BlockSpec(memory_space=pltpu.MemorySpace.VMEM)] * 9
        + [pl.BlockSpec(memory_space=pltpu.MemorySpace.SMEM)]
        + [pl.BlockSpec(memory_space=pltpu.MemorySpace.VMEM)] * 6,
        out_specs=[pl.BlockSpec(memory_space=pltpu.MemorySpace.VMEM)] * 7,
    )(sc, dx, dy, dw, dh, jnp.asarray(_WA), jnp.asarray(_HA),
      jnp.asarray(_CXA), jnp.asarray(_CYA), im_info, jnp.asarray(_SG),
      jnp.asarray(_U8), jnp.asarray(_ONES8), jnp.asarray(_M288),
      jnp.asarray(_B8), jnp.asarray(_WG))


NCHUNK = ROWS // 8
DDN = DN + NPAD


def _compact_body(rank_hbm, x1_hbm, y1_hbm, x2_hbm, y2_hbm, ar_hbm, s_hbm,
                  zeros_hbm, x1o, y1o, x2o, y2o, aro, so,
                  rnk, b0, b1, b2, b3, b4, b5, d0, d1, d2, d3, d4, d5, sem):
    cid = lax.axis_index("c")
    sid = lax.axis_index("s")
    wid = sid * 2 + cid

    denses = (d0, d1, d2, d3, d4, d5)
    for d in denses:
        pltpu.sync_copy(zeros_hbm.at[pl.ds(sid * 384, 384)],
                        d.at[pl.ds(sid * 384, 384)])
    plsc.subcore_barrier()

    for t in range(2):
        ch = wid + NW * t

        @pl.when(ch < NCHUNK)
        def _(ch=ch):
            r0 = pl.multiple_of(ch * 8, 8)
            pltpu.sync_copy(rank_hbm.at[pl.ds(r0, 8)], rnk)
            pending = []
            for d, (src, buf) in zip(denses, (
                    (x1_hbm, b0), (y1_hbm, b1), (x2_hbm, b2),
                    (y2_hbm, b3), (ar_hbm, b4), (s_hbm, b5))):
                pltpu.sync_copy(src.at[pl.ds(r0, 8)], buf)
                for j in range(8):
                    pending.append(pltpu.async_copy(
                        buf.at[j], d.at[rnk.at[j]], sem))
            for c in pending:
                c.wait()

    plsc.subcore_barrier()
    for d, dst in zip(denses, (x1o, y1o, x2o, y2o, aro, so)):
        pltpu.sync_copy(d.at[pl.ds(sid * 384, 384)],
                        dst.at[pl.ds(cid * DN + sid * 384, 384)])


@functools.lru_cache(maxsize=1)
def _compact_sc():
    return pl.kernel(
        _compact_body,
        mesh=plsc.VectorSubcoreMesh(core_axis_name="c", subcore_axis_name="s"),
        out_type=[jax.ShapeDtypeStruct((2 * DN,), jnp.float32)] * 6,
        scratch_types=[
            pltpu.VMEM((8, 128), jnp.int32)]
        + [pltpu.VMEM((8, 128), jnp.float32)] * 6
        + [pltpu.VMEM_SHARED((DDN,), jnp.float32)] * 6
        + [pltpu.SemaphoreType.DMA],
    )


def _nms_body(x1p, y1p, x2p, y2p, arp, scp, out_ref, x1_ref, y1_ref, x2_ref,
              y2_ref, ar_ref, s_ref):
    x1_ref[...] = x1p[0:ROWS2, :] + x1p[ROWS2:2 * ROWS2, :]
    y1_ref[...] = y1p[0:ROWS2, :] + y1p[ROWS2:2 * ROWS2, :]
    x2_ref[...] = x2p[0:ROWS2, :] + x2p[ROWS2:2 * ROWS2, :]
    y2_ref[...] = y2p[0:ROWS2, :] + y2p[ROWS2:2 * ROWS2, :]
    ar_ref[...] = arp[0:ROWS2, :] + arp[ROWS2:2 * ROWS2, :]
    sc_sum = scp[0:ROWS2, :] + scp[ROWS2:2 * ROWS2, :]

    ri = lax.broadcasted_iota(jnp.int32, (ROWS2, 128), 0)
    ci = lax.broadcasted_iota(jnp.int32, (ROWS2, 128), 1)
    n_i = ri * 128 + ci
    s_ref[...] = jnp.where(n_i < PRE_NMS_TOPN, sc_sum, -jnp.inf)

    nf = ri.astype(jnp.float32) * 128.0 + ci.astype(jnp.float32)
    li = lax.broadcasted_iota(jnp.int32, (1, 128), 1)

    def nms_step(step, i0):
        s = s_ref[...]
        m = jnp.max(s)
        idx = jnp.min(jnp.where(s == m, nf, jnp.float32(DN))).astype(jnp.int32)
        i0n = jnp.where(step == 0, idx, i0)
        sel = jnp.where(m == jnp.float32(NEG), i0n, idx)
        r = sel // 128
        c = sel % 128
        lm = li == c
        bx1 = jnp.sum(jnp.where(lm, x1_ref[pl.ds(r, 1), :], 0.0))
        by1 = jnp.sum(jnp.where(lm, y1_ref[pl.ds(r, 1), :], 0.0))
        bx2 = jnp.sum(jnp.where(lm, x2_ref[pl.ds(r, 1), :], 0.0))
        by2 = jnp.sum(jnp.where(lm, y2_ref[pl.ds(r, 1), :], 0.0))
        bar = jnp.sum(jnp.where(lm, ar_ref[pl.ds(r, 1), :], 0.0))
        w = jnp.maximum(0.0, jnp.minimum(bx2, x2_ref[...])
                        - jnp.maximum(bx1, x1_ref[...]) + 1.0)
        h = jnp.maximum(0.0, jnp.minimum(by2, y2_ref[...])
                        - jnp.maximum(by1, y1_ref[...]) + 1.0)
        inter = w * h
        iou = inter / (bar + ar_ref[...] - inter)
        s_ref[...] = jnp.where(iou > jnp.float32(NMS_THRESH),
                               jnp.minimum(s, jnp.float32(NEG)), s)
        rv = jnp.zeros((1, 128), jnp.float32)
        rv = jnp.where(li == 1, bx1, rv)
        rv = jnp.where(li == 2, by1, rv)
        rv = jnp.where(li == 3, bx2, rv)
        rv = jnp.where(li == 4, by2, rv)
        out_ref[pl.ds(step, 1), :] = rv
        return i0n

    lax.fori_loop(0, POST_NMS_TOPN, nms_step, jnp.int32(0))


def _nms(x1c, y1c, x2c, y2c, arc, sc):
    return pl.pallas_call(
        _nms_body,
        out_shape=jax.ShapeDtypeStruct((POST_NMS_TOPN, 128), jnp.float32),
        in_specs=[pl.BlockSpec(memory_space=pltpu.MemorySpace.VMEM)] * 6,
        out_specs=pl.BlockSpec(memory_space=pltpu.MemorySpace.VMEM),
        scratch_shapes=[pltpu.VMEM((ROWS2, 128), jnp.float32)] * 6,
    )(x1c, y1c, x2c, y2c, arc, sc)


def kernel(scores, bbox_deltas, im_info):
    sfg = jnp.transpose(scores[0, NUM_ANCHORS:], (1, 2, 0)).reshape(-1)
    dl = jnp.transpose(bbox_deltas[0], (1, 2, 0)).reshape(-1, 4)

    def pad2(v):
        return jnp.concatenate(
            [v, jnp.zeros((NPAD - N,), jnp.float32)]).reshape(ROWS, 128)

    x1, y1, x2, y2, ar, s, rank = _decode(
        pad2(sfg), pad2(dl[:, 0]), pad2(dl[:, 1]), pad2(dl[:, 2]),
        pad2(dl[:, 3]), im_info)
    x1c, y1c, x2c, y2c, arc, sc = _compact_sc()(
        rank, x1, y1, x2, y2, ar, s, jnp.zeros((DN,), jnp.float32))

    def dn(a):
        return a.reshape(2 * ROWS2, 128)

    out = _nms(dn(x1c), dn(y1c), dn(x2c), dn(y2c), dn(arc), dn(sc))
    return out[:, :5]


# speedup vs baseline: 3.4504x; 1.0216x over previous
import functools
import numpy as np
import jax
import jax.numpy as jnp
from jax import lax
from jax.experimental import pallas as pl
from jax.experimental.pallas import tpu as pltpu
from jax.experimental.pallas import tpu_sc as plsc

FEAT_STRIDE = 16
NUM_ANCHORS = 9
PRE_NMS_TOPN = 6000
POST_NMS_TOPN = 300
NMS_THRESH = 0.7
MIN_SIZE = 16.0
H, W = 50, 80
N = H * W * NUM_ANCHORS
ROWS = 288
NPAD = ROWS * 128
NEG = -1e9
DN = 6144
ROWS2 = DN // 128

NW = 32
GPW = DN // NW
VECS = NPAD // 16
NBLK = 16
VPB = VECS // NBLK


def _whctrs_k(anchor):
    w = anchor[2] - anchor[0] + 1.0
    h = anchor[3] - anchor[1] + 1.0
    return w, h, anchor[0] + 0.5 * (w - 1), anchor[1] + 0.5 * (h - 1)


def _mkanchors_k(ws, hs, x_ctr, y_ctr):
    ws = ws[:, np.newaxis]
    hs = hs[:, np.newaxis]
    return np.hstack((x_ctr - 0.5 * (ws - 1), y_ctr - 0.5 * (hs - 1),
                      x_ctr + 0.5 * (ws - 1), y_ctr + 0.5 * (hs - 1)))


def _gen_anchor_table(base_size=16, ratios=np.array([0.5, 1.0, 2.0]),
                      scales=np.array([8.0, 16.0, 32.0])):
    base_anchor = np.array([1.0, 1.0, base_size, base_size]) - 1.0
    w, h, x_ctr, y_ctr = _whctrs_k(base_anchor)
    ws0 = np.round(np.sqrt((w * h) / ratios))
    hs0 = np.round(ws0 * ratios)
    ratio_anchors = _mkanchors_k(ws0, hs0, x_ctr, y_ctr)
    outs = []
    for i in range(ratio_anchors.shape[0]):
        w, h, x_ctr, y_ctr = _whctrs_k(ratio_anchors[i, :])
        outs.append(_mkanchors_k(w * scales, h * scales, x_ctr, y_ctr))
    return np.vstack(outs).astype(np.float32)


def _anchor_consts():
    base = _gen_anchor_table().astype(np.float64)
    sx = (np.arange(W) * FEAT_STRIDE).astype(np.float64)
    sy = (np.arange(H) * FEAT_STRIDE).astype(np.float64)
    SX, SY = np.meshgrid(sx, sy)
    shifts = np.stack([SX.ravel(), SY.ravel(), SX.ravel(), SY.ravel()], axis=1)
    anchors = (base[None, :, :] + shifts[:, None, :]).reshape(-1, 4)
    widths = anchors[:, 2] - anchors[:, 0] + 1.0
    heights = anchors[:, 3] - anchors[:, 1] + 1.0
    ctr_x = anchors[:, 0] + 0.5 * widths
    ctr_y = anchors[:, 1] + 0.5 * heights

    def padr(v):
        out = np.zeros((NPAD,), np.float32)
        out[:N] = v.astype(np.float32)
        return out.reshape(ROWS, 128)

    return padr(widths), padr(heights), padr(ctr_x), padr(ctr_y)


_WA, _HA, _CXA, _CYA = _anchor_consts()


def _decode_body(sc_ref, dx_ref, dy_ref, dw_ref, dh_ref, wa_ref, ha_ref,
                 cxa_ref, cya_ref, im_ref, sg_ref, u8_ref, ones8_ref,
                 m288_ref, b8_ref, wg_ref, x1o, y1o, x2o, y2o, aro, so,
                 rank_o):
    im_h = im_ref[0, 0]
    im_w = im_ref[0, 1]
    im_scale = im_ref[0, 2]

    wa = wa_ref[...]
    ha = ha_ref[...]
    dw = jnp.clip(dw_ref[...], -10.0, 10.0)
    dh = jnp.clip(dh_ref[...], -10.0, 10.0)
    pcx = dx_ref[...] * wa + cxa_ref[...]
    pcy = dy_ref[...] * ha + cya_ref[...]
    pw = jnp.exp(dw) * wa
    ph = jnp.exp(dh) * ha
    x1 = jnp.clip(pcx - 0.5 * pw, 0.0, im_w - 1.0)
    y1 = jnp.clip(pcy - 0.5 * ph, 0.0, im_h - 1.0)
    x2 = jnp.clip(pcx + 0.5 * pw, 0.0, im_w - 1.0)
    y2 = jnp.clip(pcy + 0.5 * ph, 0.0, im_h - 1.0)
    ws = x2 - x1 + 1.0
    hs = y2 - y1 + 1.0
    min_sz = MIN_SIZE * im_scale
    keep = (ws >= min_sz) & (hs >= min_sz)

    ri = lax.broadcasted_iota(jnp.int32, (ROWS, 128), 0)
    ci = lax.broadcasted_iota(jnp.int32, (ROWS, 128), 1)
    n_i = ri * 128 + ci
    valid = n_i < N

    s0 = jnp.where(keep, sc_ref[...], jnp.float32(NEG))
    s0 = jnp.where(valid, s0, -jnp.inf)

    ks = lax.bitcast_convert_type(s0, jnp.int32)
    key_i = ks ^ ((ks >> 31) & jnp.int32(0x7FFFFFFF))
    ku = lax.bitcast_convert_type(key_i, jnp.uint32) ^ jnp.uint32(0x80000000)

    K = jnp.float32(PRE_NMS_TOPN)

    def tau_step(t, prefix):
        b = (31 - t).astype(jnp.uint32)
        cand = prefix | (jnp.uint32(1) << b)
        cnt = jnp.sum(jnp.where(ku >= cand, 1.0, 0.0))
        return jnp.where(cnt >= K, cand, prefix)

    tau = lax.fori_loop(0, 32, tau_step, jnp.uint32(0))

    c_gt = jnp.sum(jnp.where(ku > tau, 1.0, 0.0))
    needed = K - c_gt
    eqm = ku == tau

    def cut_step(t, prefix):
        cand = prefix | (jnp.int32(1) << (16 - t))
        cnt = jnp.sum(jnp.where(eqm & (n_i < cand), 1.0, 0.0))
        return jnp.where(cnt < needed, cand, prefix)

    tcut = lax.fori_loop(0, 17, cut_step, jnp.int32(0))
    elig = (ku > tau) | (eqm & (n_i <= tcut) & (needed >= 1.0))

    x1o[...] = x1
    y1o[...] = y1
    x2o[...] = x2
    y2o[...] = y2
    aro[...] = ws * hs
    so[...] = jnp.where(elig, s0, -jnp.inf)

    eligf = jnp.where(elig, 1.0, 0.0)
    hi = jax.lax.Precision.HIGHEST
    g8 = jax.lax.dot_general(eligf, sg_ref[...], (((1,), (0,)), ((), ())),
                             precision=hi)
    p_in = jax.lax.dot_general(g8, u8_ref[...], (((1,), (0,)), ((), ())),
                               precision=hi)
    rsum = jax.lax.dot_general(g8, ones8_ref[...], (((1,), (0,)), ((), ())),
                               precision=hi)
    roff = jax.lax.dot_general(m288_ref[...], rsum, (((1,), (0,)), ((), ())),
                               precision=hi)
    cum8 = roff + p_in
    cum_el = jax.lax.dot_general(cum8, b8_ref[...], (((1,), (0,)), ((), ())),
                                 precision=hi)
    p2 = jax.lax.dot_general(eligf, wg_ref[...], (((1,), (0,)), ((), ())),
                             precision=hi)
    rank = (cum_el + p2).astype(jnp.int32)
    rank_o[...] = jnp.where(elig, rank, jnp.int32(DN) + n_i)


def _sel_consts():
    sg = np.zeros((128, 8), np.float32)
    for l in range(128):
        sg[l, l // 16] = 1.0
    u8 = np.triu(np.ones((8, 8), np.float32), 1)
    ones8 = np.ones((8, 1), np.float32)
    m288 = np.tril(np.ones((288, 288), np.float32), -1)
    b8 = np.zeros((8, 128), np.float32)
    for l in range(128):
        b8[l // 16, l] = 1.0
    wg = np.zeros((128, 128), np.float32)
    for a in range(128):
        for b in range(128):
            if a // 16 == b // 16 and a < b:
                wg[a, b] = 1.0
    return sg, u8, ones8, m288, b8, wg


_SG, _U8, _ONES8, _M288, _B8, _WG = _sel_consts()


def _decode(sc, dx, dy, dw, dh, im_info):
    shp = jax.ShapeDtypeStruct((ROWS, 128), jnp.float32)
    shpi = jax.ShapeDtypeStruct((ROWS, 128), jnp.int32)
    return pl.pallas_call(
        _decode_body,
        out_shape=[shp] * 6 + [shpi],
        in_specs=[pl.BlockSpec(memory_space=pltpu.MemorySpace.VMEM)] * 9
        + [pl.BlockSpec(memory_space=pltpu.MemorySpace.SMEM)]
        + [pl.BlockSpec(memory_space=pltpu.MemorySpace.VMEM)] * 6,
        out_specs=[pl.BlockSpec(memory_space=pltpu.MemorySpace.VMEM)] * 7,
    )(sc, dx, dy, dw, dh, jnp.asarray(_WA), jnp.asarray(_HA),
      jnp.asarray(_CXA), jnp.asarray(_CYA), im_info, jnp.asarray(_SG),
      jnp.asarray(_U8), jnp.asarray(_ONES8), jnp.asarray(_M288),
      jnp.asarray(_B8), jnp.asarray(_WG))


NCHUNK = ROWS // 8
DDN = DN + NPAD


def _compact_body(rank_hbm, x1_hbm, y1_hbm, x2_hbm, y2_hbm, ar_hbm, s_hbm,
                  zeros_hbm, x1o, y1o, x2o, y2o, aro, so,
                  rnk, b0, b1, b2, b3, b4, b5, d0, d1, d2, d3, d4, d5, sem):
    cid = lax.axis_index("c")
    sid = lax.axis_index("s")
    wid = sid * 2 + cid

    denses = (d0, d1, d2, d3, d4, d5)
    for d in denses:
        pltpu.sync_copy(zeros_hbm.at[pl.ds(sid * 384, 384)],
                        d.at[pl.ds(sid * 384, 384)])
    plsc.subcore_barrier()

    for t in range(2):
        ch = wid + NW * t

        @pl.when(ch < NCHUNK)
        def _(ch=ch):
            r0 = pl.multiple_of(ch * 8, 8)
            pltpu.sync_copy(rank_hbm.at[pl.ds(r0, 8)], rnk)
            pending = []
            for d, (src, buf) in zip(denses, (
                    (x1_hbm, b0), (y1_hbm, b1), (x2_hbm, b2),
                    (y2_hbm, b3), (ar_hbm, b4), (s_hbm, b5))):
                pltpu.sync_copy(src.at[pl.ds(r0, 8)], buf)
                for j in range(8):
                    pending.append(pltpu.async_copy(
                        buf.at[j], d.at[rnk.at[j]], sem))
            for c in pending:
                c.wait()

    plsc.subcore_barrier()
    for d, dst in zip(denses, (x1o, y1o, x2o, y2o, aro, so)):
        pltpu.sync_copy(d.at[pl.ds(sid * 384, 384)],
                        dst.at[pl.ds(cid * DN + sid * 384, 384)])


@functools.lru_cache(maxsize=1)
def _compact_sc():
    return pl.kernel(
        _compact_body,
        mesh=plsc.VectorSubcoreMesh(core_axis_name="c", subcore_axis_name="s"),
        out_type=[jax.ShapeDtypeStruct((2 * DN,), jnp.float32)] * 6,
        scratch_types=[
            pltpu.VMEM((8, 128), jnp.int32)]
        + [pltpu.VMEM((8, 128), jnp.float32)] * 6
        + [pltpu.VMEM_SHARED((DDN,), jnp.float32)] * 6
        + [pltpu.SemaphoreType.DMA],
    )


def _nms_body(x1p, y1p, x2p, y2p, arp, scp, out_ref, x1_ref, y1_ref, x2_ref,
              y2_ref, ar_ref, s_ref):
    x1_ref[...] = x1p[0:ROWS2, :] + x1p[ROWS2:2 * ROWS2, :]
    y1_ref[...] = y1p[0:ROWS2, :] + y1p[ROWS2:2 * ROWS2, :]
    x2_ref[...] = x2p[0:ROWS2, :] + x2p[ROWS2:2 * ROWS2, :]
    y2_ref[...] = y2p[0:ROWS2, :] + y2p[ROWS2:2 * ROWS2, :]
    ar_ref[...] = arp[0:ROWS2, :] + arp[ROWS2:2 * ROWS2, :]
    sc_sum = scp[0:ROWS2, :] + scp[ROWS2:2 * ROWS2, :]

    ri = lax.broadcasted_iota(jnp.int32, (ROWS2, 128), 0)
    ci = lax.broadcasted_iota(jnp.int32, (ROWS2, 128), 1)
    n_i = ri * 128 + ci
    s0 = jnp.where(n_i < PRE_NMS_TOPN, sc_sum, -jnp.inf)
    s_ref[...] = s0

    nf = ri.astype(jnp.float32) * 128.0 + ci.astype(jnp.float32)
    li = lax.broadcasted_iota(jnp.int32, (1, 128), 1)

    m0 = jnp.max(s0)
    idx0 = jnp.argmax(s0.reshape(-1)).astype(jnp.int32)

    def nms_step(step, carry):
        m, idx, i0 = carry
        i0n = jnp.where(step == 0, idx, i0)
        sel = jnp.where(m == jnp.float32(NEG), i0n, idx)
        r = sel // 128
        c = sel % 128
        lm = li == c
        bx1 = jnp.sum(jnp.where(lm, x1_ref[pl.ds(r, 1), :], 0.0))
        by1 = jnp.sum(jnp.where(lm, y1_ref[pl.ds(r, 1), :], 0.0))
        bx2 = jnp.sum(jnp.where(lm, x2_ref[pl.ds(r, 1), :], 0.0))
        by2 = jnp.sum(jnp.where(lm, y2_ref[pl.ds(r, 1), :], 0.0))
        bar = jnp.sum(jnp.where(lm, ar_ref[pl.ds(r, 1), :], 0.0))
        s = s_ref[...]
        w = jnp.maximum(0.0, jnp.minimum(bx2, x2_ref[...])
                        - jnp.maximum(bx1, x1_ref[...]) + 1.0)
        h = jnp.maximum(0.0, jnp.minimum(by2, y2_ref[...])
                        - jnp.maximum(by1, y1_ref[...]) + 1.0)
        inter = w * h
        iou = inter / (bar + ar_ref[...] - inter)
        s_new = jnp.where(iou > jnp.float32(NMS_THRESH),
                          jnp.minimum(s, jnp.float32(NEG)), s)
        s_ref[...] = s_new
        m2 = jnp.max(s_new)
        idx2 = jnp.argmax(s_new.reshape(-1)).astype(jnp.int32)
        rv = jnp.zeros((1, 128), jnp.float32)
        rv = jnp.where(li == 1, bx1, rv)
        rv = jnp.where(li == 2, by1, rv)
        rv = jnp.where(li == 3, bx2, rv)
        rv = jnp.where(li == 4, by2, rv)
        out_ref[pl.ds(step, 1), :] = rv
        return m2, idx2, i0n

    lax.fori_loop(0, POST_NMS_TOPN, nms_step, (m0, idx0, jnp.int32(0)))


def _nms(x1c, y1c, x2c, y2c, arc, sc):
    return pl.pallas_call(
        _nms_body,
        out_shape=jax.ShapeDtypeStruct((POST_NMS_TOPN, 128), jnp.float32),
        in_specs=[pl.BlockSpec(memory_space=pltpu.MemorySpace.VMEM)] * 6,
        out_specs=pl.BlockSpec(memory_space=pltpu.MemorySpace.VMEM),
        scratch_shapes=[pltpu.VMEM((ROWS2, 128), jnp.float32)] * 6,
    )(x1c, y1c, x2c, y2c, arc, sc)


def kernel(scores, bbox_deltas, im_info):
    sfg = jnp.transpose(scores[0, NUM_ANCHORS:], (1, 2, 0)).reshape(-1)
    dl = jnp.transpose(bbox_deltas[0], (1, 2, 0)).reshape(-1, 4)

    def pad2(v):
        return jnp.concatenate(
            [v, jnp.zeros((NPAD - N,), jnp.float32)]).reshape(ROWS, 128)

    x1, y1, x2, y2, ar, s, rank = _decode(
        pad2(sfg), pad2(dl[:, 0]), pad2(dl[:, 1]), pad2(dl[:, 2]),
        pad2(dl[:, 3]), im_info)
    x1c, y1c, x2c, y2c, arc, sc = _compact_sc()(
        rank, x1, y1, x2, y2, ar, s, jnp.zeros((DN,), jnp.float32))

    def dn(a):
        return a.reshape(2 * ROWS2, 128)

    out = _nms(dn(x1c), dn(y1c), dn(x2c), dn(y2c), dn(arc), dn(sc))
    return out[:, :5]
